# baseline (speedup 1.0000x reference)
import functools

import jax
import jax.numpy as jnp
from jax import lax
from jax.experimental import pallas as pl
from jax.experimental.pallas import tpu as pltpu
from jax.experimental.pallas import tpu_sc as plsc

B = 16384
D = 128
NW = 32
PER_W = B // NW
CHUNK = 128
NCHUNK = PER_W // CHUNK
L = 16


def _body(w_hbm, h_hbm, uidx_hbm, vidx_hbm, out_hbm,
          uidx_v, vidx_v, u_rows, v_rows, out_v, sem_u, sem_v):
    wid = lax.axis_index("s") * 2 + lax.axis_index("c")
    base = wid * PER_W

    pltpu.sync_copy(uidx_hbm.at[wid], uidx_v)
    pltpu.sync_copy(vidx_hbm.at[wid], vidx_v)

    lane = lax.iota(jnp.int32, L)
    for c in range(NCHUNK):
        cu = pltpu.async_copy(w_hbm.at[uidx_v.at[c]], u_rows, sem_u)
        cv = pltpu.async_copy(h_hbm.at[vidx_v.at[c]], v_rows, sem_v)
        cu.wait()
        cv.wait()

        for g in range(CHUNK // L):
            rows = lane + g * L

            def col_body(j, acc):
                cols = jnp.full((L,), j, dtype=jnp.int32)
                u = plsc.load_gather(u_rows, [rows, cols])
                v = plsc.load_gather(v_rows, [rows, cols])
                return acc + u * v

            acc = lax.fori_loop(0, D, col_body, jnp.zeros((L,), jnp.float32))
            out_v[pl.ds(c * CHUNK + g * L, L)] = acc

    def sig_body(g, carry):
        z = out_v[pl.ds(g * L, L)]
        out_v[pl.ds(g * L, L)] = 1.0 / (1.0 + jnp.exp(-z))
        return carry

    lax.fori_loop(0, PER_W // L, sig_body, 0)

    pltpu.sync_copy(out_v, out_hbm.at[pl.ds(base, PER_W)])


@jax.jit
def _run(W, H, uidx, vidx):
    mesh = plsc.VectorSubcoreMesh(core_axis_name="c", subcore_axis_name="s")
    kfn = pl.kernel(
        _body,
        mesh=mesh,
        out_type=jax.ShapeDtypeStruct((B,), jnp.float32),
        scratch_types=[
            pltpu.VMEM((NCHUNK, CHUNK), jnp.int32),
            pltpu.VMEM((NCHUNK, CHUNK), jnp.int32),
            pltpu.VMEM((CHUNK, D), jnp.float32),
            pltpu.VMEM((CHUNK, D), jnp.float32),
            pltpu.VMEM((PER_W,), jnp.float32),
            pltpu.SemaphoreType.DMA,
            pltpu.SemaphoreType.DMA,
        ],
        compiler_params=pltpu.CompilerParams(needs_layout_passes=False),
    )
    return kfn(W, H, uidx, vidx)


def kernel(x, W, H):
    xi = x.astype(jnp.int32)
    uidx = xi[:, 0].reshape(NW, NCHUNK, CHUNK)
    vidx = xi[:, 1].reshape(NW, NCHUNK, CHUNK)
    return _run(W, H, uidx, vidx)


# speedup vs baseline: 1.1280x; 1.1280x over previous
import jax
import jax.numpy as jnp
from jax import lax
from jax.experimental import pallas as pl
from jax.experimental.pallas import tpu as pltpu
from jax.experimental.pallas import tpu_sc as plsc

B = 16384
D = 128
NW = 32
PER_W = B // NW
CHUNK = 128
NCHUNK = PER_W // CHUNK
L = 16
UNROLL = 16


def _body(w_hbm, h_hbm, uidx_hbm, vidx_hbm, out_hbm,
          uidx_v, vidx_v, u0, v0, u1, v1, out_v,
          su0, sv0, su1, sv1):
    wid = lax.axis_index("s") * 2 + lax.axis_index("c")
    base = wid * PER_W

    pltpu.sync_copy(uidx_hbm.at[wid], uidx_v)
    pltpu.sync_copy(vidx_hbm.at[wid], vidx_v)

    bufs = [(u0, v0, su0, sv0), (u1, v1, su1, sv1)]

    def start(c):
        ub, vb, su, sv = bufs[c % 2]
        cu = pltpu.async_copy(w_hbm.at[uidx_v.at[c]], ub, su)
        cv = pltpu.async_copy(h_hbm.at[vidx_v.at[c]], vb, sv)
        return cu, cv

    lane = lax.iota(jnp.int32, L)
    zero = jnp.zeros((L,), jnp.float32)
    pending = start(0)

    for c in range(NCHUNK):
        u_rows, v_rows = bufs[c % 2][0], bufs[c % 2][1]
        cu, cv = pending
        if c + 1 < NCHUNK:
            pending = start(c + 1)
        cu.wait()
        cv.wait()

        def group_body(g, carry, u_rows=u_rows, v_rows=v_rows, c=c):
            rows = lane + g * L

            def jblock(i, accs):
                a0, a1, a2, a3 = accs
                jb = i * UNROLL
                for jj in range(UNROLL):
                    cols = jnp.full((L,), jb + jj, dtype=jnp.int32)
                    p = (plsc.load_gather(u_rows, [rows, cols])
                         * plsc.load_gather(v_rows, [rows, cols]))
                    if jj % 4 == 0:
                        a0 = a0 + p
                    elif jj % 4 == 1:
                        a1 = a1 + p
                    elif jj % 4 == 2:
                        a2 = a2 + p
                    else:
                        a3 = a3 + p
                return a0, a1, a2, a3

            a0, a1, a2, a3 = lax.fori_loop(
                0, D // UNROLL, jblock, (zero, zero, zero, zero))
            out_v[pl.ds(c * CHUNK + g * L, L)] = (a0 + a1) + (a2 + a3)
            return carry

        lax.fori_loop(0, CHUNK // L, group_body, 0)

    def sig_body(g, carry):
        z = out_v[pl.ds(g * L, L)]
        out_v[pl.ds(g * L, L)] = 1.0 / (1.0 + jnp.exp(-z))
        return carry

    lax.fori_loop(0, PER_W // L, sig_body, 0)

    pltpu.sync_copy(out_v, out_hbm.at[pl.ds(base, PER_W)])


@jax.jit
def _run(W, H, uidx, vidx):
    mesh = plsc.VectorSubcoreMesh(core_axis_name="c", subcore_axis_name="s")
    kfn = pl.kernel(
        _body,
        mesh=mesh,
        out_type=jax.ShapeDtypeStruct((B,), jnp.float32),
        scratch_types=[
            pltpu.VMEM((NCHUNK, CHUNK), jnp.int32),
            pltpu.VMEM((NCHUNK, CHUNK), jnp.int32),
            pltpu.VMEM((CHUNK, D), jnp.float32),
            pltpu.VMEM((CHUNK, D), jnp.float32),
            pltpu.VMEM((CHUNK, D), jnp.float32),
            pltpu.VMEM((CHUNK, D), jnp.float32),
            pltpu.VMEM((PER_W,), jnp.float32),
            pltpu.SemaphoreType.DMA,
            pltpu.SemaphoreType.DMA,
            pltpu.SemaphoreType.DMA,
            pltpu.SemaphoreType.DMA,
        ],
        compiler_params=pltpu.CompilerParams(needs_layout_passes=False),
    )
    return kfn(W, H, uidx, vidx)


def kernel(x, W, H):
    xi = x.astype(jnp.int32)
    uidx = xi[:, 0].reshape(NW, NCHUNK, CHUNK)
    vidx = xi[:, 1].reshape(NW, NCHUNK, CHUNK)
    return _run(W, H, uidx, vidx)


# speedup vs baseline: 2.9910x; 2.6516x over previous
import jax
import jax.numpy as jnp
from jax import lax
from jax.experimental import pallas as pl
from jax.experimental.pallas import tpu as pltpu
from jax.experimental.pallas import tpu_sc as plsc

B = 16384
D = 128
NW = 32
PER_W = B // NW
CHUNK = 128
NCHUNK = PER_W // CHUNK
L = 16
UNROLL = 16


def _body(w_hbm, h_hbm, uidx_hbm, vidx_hbm, out_hbm,
          uidx_v, vidx_v, u0, v0, u1, v1, out_v,
          su0, sv0, su1, sv1):
    wid = lax.axis_index("s") * 2 + lax.axis_index("c")
    base = wid * PER_W

    pltpu.sync_copy(uidx_hbm.at[wid], uidx_v)
    pltpu.sync_copy(vidx_hbm.at[wid], vidx_v)

    bufs = [(u0, v0, su0, sv0), (u1, v1, su1, sv1)]

    def start(c):
        ub, vb, su, sv = bufs[c % 2]
        cu = pltpu.async_copy(w_hbm.at[uidx_v.at[c]], ub, su)
        cv = pltpu.async_copy(h_hbm.at[vidx_v.at[c]], vb, sv)
        return cu, cv

    lane = lax.iota(jnp.int32, L)
    zero = jnp.zeros((L,), jnp.float32)
    pending = start(0)

    for c in range(NCHUNK):
        u_rows, v_rows = bufs[c % 2][0], bufs[c % 2][1]
        cu, cv = pending
        if c + 1 < NCHUNK:
            pending = start(c + 1)
        cu.wait()
        cv.wait()

        def group_body(g, carry, u_rows=u_rows, v_rows=v_rows, c=c):
            rows = lane + g * L

            def jblock(i, accs):
                a0, a1, a2, a3 = accs
                colbase = lane + i * UNROLL
                for jj in range(UNROLL):
                    cols = (colbase + jj) & (D - 1)
                    p = (plsc.load_gather(u_rows, [rows, cols])
                         * plsc.load_gather(v_rows, [rows, cols]))
                    if jj % 4 == 0:
                        a0 = a0 + p
                    elif jj % 4 == 1:
                        a1 = a1 + p
                    elif jj % 4 == 2:
                        a2 = a2 + p
                    else:
                        a3 = a3 + p
                return a0, a1, a2, a3

            a0, a1, a2, a3 = lax.fori_loop(
                0, D // UNROLL, jblock, (zero, zero, zero, zero))
            out_v[pl.ds(c * CHUNK + g * L, L)] = (a0 + a1) + (a2 + a3)
            return carry

        lax.fori_loop(0, CHUNK // L, group_body, 0)

    def sig_body(g, carry):
        z = out_v[pl.ds(g * L, L)]
        out_v[pl.ds(g * L, L)] = 1.0 / (1.0 + jnp.exp(-z))
        return carry

    lax.fori_loop(0, PER_W // L, sig_body, 0)

    pltpu.sync_copy(out_v, out_hbm.at[pl.ds(base, PER_W)])


@jax.jit
def _run(W, H, uidx, vidx):
    mesh = plsc.VectorSubcoreMesh(core_axis_name="c", subcore_axis_name="s")
    kfn = pl.kernel(
        _body,
        mesh=mesh,
        out_type=jax.ShapeDtypeStruct((B,), jnp.float32),
        scratch_types=[
            pltpu.VMEM((NCHUNK, CHUNK), jnp.int32),
            pltpu.VMEM((NCHUNK, CHUNK), jnp.int32),
            pltpu.VMEM((CHUNK, D), jnp.float32),
            pltpu.VMEM((CHUNK, D), jnp.float32),
            pltpu.VMEM((CHUNK, D), jnp.float32),
            pltpu.VMEM((CHUNK, D), jnp.float32),
            pltpu.VMEM((PER_W,), jnp.float32),
            pltpu.SemaphoreType.DMA,
            pltpu.SemaphoreType.DMA,
            pltpu.SemaphoreType.DMA,
            pltpu.SemaphoreType.DMA,
        ],
        compiler_params=pltpu.CompilerParams(needs_layout_passes=False),
    )
    return kfn(W, H, uidx, vidx)


def kernel(x, W, H):
    xi = x.astype(jnp.int32)
    uidx = xi[:, 0].reshape(NW, NCHUNK, CHUNK)
    vidx = xi[:, 1].reshape(NW, NCHUNK, CHUNK)
    return _run(W, H, uidx, vidx)
